# baseline (speedup 1.0000x reference)
import functools

import jax
import jax.numpy as jnp
from jax.experimental import pallas as pl
from jax.experimental.pallas import tpu as pltpu

NUM_GROUP_K = 256
GROUP_SIZE_K = 32
B_K = 16
N_K = 8192


def _fps_body(xt_ref, ct_ref):
    x = xt_ref[0]
    y = xt_ref[1]
    z = xt_ref[2]
    nb = x.shape[0]
    lane = jax.lax.broadcasted_iota(jnp.int32, (nb, N_K), 1)
    gcol = jax.lax.broadcasted_iota(jnp.int32, (nb, NUM_GROUP_K), 1)

    def body(i, carry):
        dists, far, cxa, cya, cza = carry
        mask = lane == far
        cx = jnp.sum(jnp.where(mask, x, 0.0), axis=1, keepdims=True)
        cy = jnp.sum(jnp.where(mask, y, 0.0), axis=1, keepdims=True)
        cz = jnp.sum(jnp.where(mask, z, 0.0), axis=1, keepdims=True)
        sel = gcol == i
        cxa = jnp.where(sel, cx, cxa)
        cya = jnp.where(sel, cy, cya)
        cza = jnp.where(sel, cz, cza)
        dx = x - cx
        dy = y - cy
        dz = z - cz
        d = dx * dx + dy * dy + dz * dz
        dists = jnp.minimum(dists, d)
        m = jnp.max(dists, axis=1, keepdims=True)
        far = jnp.min(
            jnp.where(dists == m, lane, jnp.int32(N_K)), axis=1, keepdims=True
        )
        return dists, far, cxa, cya, cza

    init = (
        jnp.full((nb, N_K), 1e10, dtype=jnp.float32),
        jnp.zeros((nb, 1), dtype=jnp.int32),
        jnp.zeros((nb, NUM_GROUP_K), dtype=jnp.float32),
        jnp.zeros((nb, NUM_GROUP_K), dtype=jnp.float32),
        jnp.zeros((nb, NUM_GROUP_K), dtype=jnp.float32),
    )
    _, _, cxa, cya, cza = jax.lax.fori_loop(0, NUM_GROUP_K, body, init)
    ct_ref[0] = cxa
    ct_ref[1] = cya
    ct_ref[2] = cza


@functools.partial(jax.jit, static_argnames=("interpret",))
def _fps_centers(xt, interpret=False):
    return pl.pallas_call(
        _fps_body,
        grid=(2,),
        in_specs=[pl.BlockSpec((3, B_K // 2, N_K), lambda i: (0, i, 0))],
        out_specs=pl.BlockSpec((3, B_K // 2, NUM_GROUP_K), lambda i: (0, i, 0)),
        out_shape=jax.ShapeDtypeStruct((3, B_K, NUM_GROUP_K), jnp.float32),
        compiler_params=pltpu.CompilerParams(
            dimension_semantics=("parallel",),
        ),
        interpret=interpret,
    )(xt)


def _d2_body(xt_ref, c_ref, d2_ref, v256_ref):
    x = xt_ref[0, 0]
    y = xt_ref[1, 0]
    z = xt_ref[2, 0]
    cx = c_ref[0, :, 0:1]
    cy = c_ref[0, :, 1:2]
    cz = c_ref[0, :, 2:3]
    dx = cx - x
    dy = cy - y
    dz = cz - z
    d2 = dx * dx + dy * dy + dz * dz
    d2_ref[0] = d2
    v = d2
    for w in (4096, 2048, 1024, 512, 256, 128):
        v = jnp.minimum(v[:, :w], v[:, w : 2 * w])
    v256_ref[0] = v


@functools.partial(jax.jit, static_argnames=("interpret",))
def _d2_v256(xt, center, interpret=False):
    return pl.pallas_call(
        _d2_body,
        grid=(B_K,),
        in_specs=[
            pl.BlockSpec((3, 1, 1, N_K), lambda i: (0, i, 0, 0)),
            pl.BlockSpec((1, NUM_GROUP_K, 3), lambda i: (i, 0, 0)),
        ],
        out_specs=[
            pl.BlockSpec((1, NUM_GROUP_K, N_K), lambda i: (i, 0, 0)),
            pl.BlockSpec((1, NUM_GROUP_K, 128), lambda i: (i, 0, 0)),
        ],
        out_shape=[
            jax.ShapeDtypeStruct((B_K, NUM_GROUP_K, N_K), jnp.float32),
            jax.ShapeDtypeStruct((B_K, NUM_GROUP_K, 128), jnp.float32),
        ],
        compiler_params=pltpu.CompilerParams(
            dimension_semantics=("parallel",),
        ),
        interpret=interpret,
    )(xt[:, :, None, :], center)


from jax import lax
from jax.experimental.pallas import tpu_sc as plsc

_CAP = 4096
_BIG = 1 << 30
_INF = float("inf")


def _iota16():
    return lax.broadcasted_iota(jnp.int32, (16,), 0)


def _splat(v, dtype=jnp.int32):
    return jnp.full((16,), v, dtype=dtype)


def _bcast_min(v, scr):
    for s in (8, 4, 2, 1):
        scr[...] = v
        rot = plsc.load_gather(scr, [(_iota16() + s) & 15])
        v = jnp.minimum(v, rot)
    return v


def _sc_group_body(
    d2_hbm, v256_hbm, xq_hbm, ctr_hbm, out_hbm,
    d2bufA, d2bufB, vbufA, vbufB, xqbuf, ctrbuf,
    scrf0, scri0, scrf1, scri1, selidsA, selidsB, outbuf,
    sem0, sem1, sem2, sem3,
):
    cid = lax.axis_index("c")
    sid = lax.axis_index("s")
    wid = sid * 2 + cid
    b = wid // 2
    half = wid % 2
    row0 = half * 128

    pltpu.sync_copy(xq_hbm.at[b], xqbuf)
    coff = pl.multiple_of(half * 384, 384)
    pltpu.sync_copy(ctr_hbm.at[b, pl.ds(coff, 384)], ctrbuf)

    sems = (sem0, sem1, sem2, sem3)
    dbufs = (d2bufA, d2bufB)
    vbufs = (vbufA, vbufB)

    def issue(r, side, bk):
        sem = sems[2 * bk + side]
        pltpu.async_copy(d2_hbm.at[b, row0 + r], dbufs[side].at[bk], sem)
        pltpu.async_copy(v256_hbm.at[b, row0 + r], vbufs[side].at[bk], sem)

    def wait(r, side, bk):
        sem = sems[2 * bk + side]
        pltpu.make_async_copy(
            d2_hbm.at[b, row0 + r], dbufs[side].at[bk], sem
        ).wait()
        pltpu.make_async_copy(
            v256_hbm.at[b, row0 + r], vbufs[side].at[bk], sem
        ).wait()

    def process_pair(r0, bk):
        lane0 = _iota16() == 0
        inf16 = _splat(_INF, jnp.float32)
        rows = ((r0, 0), (r0 + 1, 1))

        def one_row(k, r, db, vb, selids, p, scrf, scri):
            slot16 = _splat(bk)
            cm = [vb[bk, pl.ds(16 * j, 16)] for j in range(8)]
            vm = cm[0]
            for j in range(1, 8):
                vm = jnp.minimum(vm, cm[j])
            m = _bcast_min(vm, scrf)

            acc = _splat(_BIG)
            for j in range(8):
                acc = jnp.minimum(
                    acc, jnp.where(cm[j] == m, _iota16() + 16 * j, _BIG)
                )
            cls = _bcast_min(acc, scri)

            idxs = [cls + 128 * (_iota16() + 16 * t) for t in range(4)]
            mvs = [plsc.load_gather(db, [slot16, ix]) for ix in idxs]
            p_acc = jnp.where(mvs[0] == m, idxs[0], _BIG)
            for t in range(1, 4):
                p_acc = jnp.minimum(
                    p_acc, jnp.where(mvs[t] == m, idxs[t], _BIG)
                )
            pid = _bcast_min(p_acc, scri)

            plsc.store_scatter(selids, [_splat(k)], pid, mask=lane0)
            plsc.store_scatter(db, [slot16, pid], inf16, mask=lane0)
            nmv = jnp.where(idxs[0] == pid, inf16, mvs[0])
            for t in range(1, 4):
                nmv = jnp.minimum(nmv, jnp.where(idxs[t] == pid, inf16, mvs[t]))
            nm = _bcast_min(nmv, scrf)
            plsc.store_scatter(vb, [slot16, cls], nm, mask=lane0)

        def sel_body(k, _):
            for (r, p), db, vb, sel, sf, si in zip(
                rows, dbufs, vbufs, (selidsA, selidsB),
                (scrf0, scrf1), (scri0, scri1),
            ):
                one_row(k, r, db, vb, sel, p, sf, si)
            return 0

        lax.fori_loop(0, GROUP_SIZE_K, sel_body, 0)

        for (r, p), sel in zip(rows, (selidsA, selidsB)):
            for kk in range(2):
                kid = sel[pl.ds(16 * kk, 16)]
                for ch in range(6):
                    val = plsc.load_gather(xqbuf, [_splat(ch), kid])
                    if ch < 3:
                        ctr = plsc.load_gather(ctrbuf, [_splat(r * 3 + ch)])
                        val = val - ctr
                    pos = (_iota16() + 16 * kk) * 6 + ch
                    plsc.store_scatter(outbuf, [_splat(p), pos], val)
        pltpu.sync_copy(outbuf, out_hbm.at[b, pl.ds(row0 + r0, 2)])

    def issue_pair(r0, bank):
        issue(r0, 0, bank)
        issue(r0 + 1, 1, bank)

    def wait_pair(r0, bank):
        wait(r0, 0, bank)
        wait(r0 + 1, 1, bank)

    issue_pair(0, 0)

    def iter2(i, _):
        r0 = 4 * i
        issue_pair(r0 + 2, 1)
        wait_pair(r0, 0)
        process_pair(r0, 0)

        @pl.when(r0 + 4 < 128)
        def _():
            issue_pair(r0 + 4, 0)

        wait_pair(r0 + 2, 1)
        process_pair(r0 + 2, 1)
        return 0

    lax.fori_loop(0, 32, iter2, 0)


_SC_SCRATCH = [
    pltpu.VMEM((2, N_K), jnp.float32),
    pltpu.VMEM((2, N_K), jnp.float32),
    pltpu.VMEM((2, 128), jnp.float32),
    pltpu.VMEM((2, 128), jnp.float32),
    pltpu.VMEM((6, N_K), jnp.float32),
    pltpu.VMEM((384,), jnp.float32),
    pltpu.VMEM((16,), jnp.float32),
    pltpu.VMEM((16,), jnp.int32),
    pltpu.VMEM((16,), jnp.float32),
    pltpu.VMEM((16,), jnp.int32),
    pltpu.VMEM((32,), jnp.int32),
    pltpu.VMEM((32,), jnp.int32),
    pltpu.VMEM((2, 192), jnp.float32),
    pltpu.SemaphoreType.DMA,
    pltpu.SemaphoreType.DMA,
    pltpu.SemaphoreType.DMA,
    pltpu.SemaphoreType.DMA,
]


@functools.lru_cache(maxsize=None)
def _sc_group():
    return pl.kernel(
        _sc_group_body,
        out_type=jax.ShapeDtypeStruct((B_K, NUM_GROUP_K, 192), jnp.float32),
        mesh=plsc.VectorSubcoreMesh(core_axis_name="c", subcore_axis_name="s"),
        scratch_types=_SC_SCRATCH,
        compiler_params=pltpu.CompilerParams(needs_layout_passes=False),
    )


def kernel(xyz):
    B, N, C = xyz.shape
    xyz_only = xyz[:, :, :3]

    xt = jnp.transpose(xyz_only, (2, 0, 1))
    ct = _fps_centers(xt)
    center = jnp.transpose(ct, (1, 2, 0))

    d2, v256 = _d2_v256(xt, center)

    xq = jnp.transpose(xyz, (0, 2, 1))
    ctr_flat = center.reshape(B, NUM_GROUP_K * 3)
    neigh = _sc_group()(d2, v256, xq, ctr_flat)
    neighborhood = neigh.reshape(B, NUM_GROUP_K, GROUP_SIZE_K, 6)
    return (neighborhood, center)


# speedup vs baseline: 1.0404x; 1.0404x over previous
import functools

import jax
import jax.numpy as jnp
from jax.experimental import pallas as pl
from jax.experimental.pallas import tpu as pltpu

NUM_GROUP_K = 256
GROUP_SIZE_K = 32
B_K = 16
N_K = 8192


def _fps_body(xt_ref, ct_ref):
    x = xt_ref[0]
    y = xt_ref[1]
    z = xt_ref[2]
    nb = x.shape[0]
    lane = jax.lax.broadcasted_iota(jnp.int32, (nb, N_K), 1)
    gcol = jax.lax.broadcasted_iota(jnp.int32, (nb, NUM_GROUP_K), 1)

    def body(i, carry):
        dists, far, cxa, cya, cza = carry
        mask = lane == far
        cx = jnp.sum(jnp.where(mask, x, 0.0), axis=1, keepdims=True)
        cy = jnp.sum(jnp.where(mask, y, 0.0), axis=1, keepdims=True)
        cz = jnp.sum(jnp.where(mask, z, 0.0), axis=1, keepdims=True)
        sel = gcol == i
        cxa = jnp.where(sel, cx, cxa)
        cya = jnp.where(sel, cy, cya)
        cza = jnp.where(sel, cz, cza)
        dx = x - cx
        dy = y - cy
        dz = z - cz
        d = dx * dx + dy * dy + dz * dz
        dists = jnp.minimum(dists, d)
        m = jnp.max(dists, axis=1, keepdims=True)
        far = jnp.min(
            jnp.where(dists == m, lane, jnp.int32(N_K)), axis=1, keepdims=True
        )
        return dists, far, cxa, cya, cza

    init = (
        jnp.full((nb, N_K), 1e10, dtype=jnp.float32),
        jnp.zeros((nb, 1), dtype=jnp.int32),
        jnp.zeros((nb, NUM_GROUP_K), dtype=jnp.float32),
        jnp.zeros((nb, NUM_GROUP_K), dtype=jnp.float32),
        jnp.zeros((nb, NUM_GROUP_K), dtype=jnp.float32),
    )
    _, _, cxa, cya, cza = jax.lax.fori_loop(0, NUM_GROUP_K, body, init)
    ct_ref[0] = cxa
    ct_ref[1] = cya
    ct_ref[2] = cza


@functools.partial(jax.jit, static_argnames=("interpret",))
def _fps_centers(xt, interpret=False):
    return pl.pallas_call(
        _fps_body,
        grid=(2,),
        in_specs=[pl.BlockSpec((3, B_K // 2, N_K), lambda i: (0, i, 0))],
        out_specs=pl.BlockSpec((3, B_K // 2, NUM_GROUP_K), lambda i: (0, i, 0)),
        out_shape=jax.ShapeDtypeStruct((3, B_K, NUM_GROUP_K), jnp.float32),
        compiler_params=pltpu.CompilerParams(
            dimension_semantics=("parallel",),
        ),
        interpret=interpret,
    )(xt)


def _d2_body(xt_ref, c_ref, d2_ref, v256_ref):
    x = xt_ref[0, 0]
    y = xt_ref[1, 0]
    z = xt_ref[2, 0]
    cx = c_ref[0, :, 0:1]
    cy = c_ref[0, :, 1:2]
    cz = c_ref[0, :, 2:3]
    dx = cx - x
    dy = cy - y
    dz = cz - z
    d2 = dx * dx + dy * dy + dz * dz
    d2_ref[0] = d2
    v = d2
    for w in (4096, 2048, 1024, 512, 256):
        v = jnp.minimum(v[:, :w], v[:, w : 2 * w])
    v256_ref[0] = v


@functools.partial(jax.jit, static_argnames=("interpret",))
def _d2_v256(xt, center, interpret=False):
    return pl.pallas_call(
        _d2_body,
        grid=(B_K,),
        in_specs=[
            pl.BlockSpec((3, 1, 1, N_K), lambda i: (0, i, 0, 0)),
            pl.BlockSpec((1, NUM_GROUP_K, 3), lambda i: (i, 0, 0)),
        ],
        out_specs=[
            pl.BlockSpec((1, NUM_GROUP_K, N_K), lambda i: (i, 0, 0)),
            pl.BlockSpec((1, NUM_GROUP_K, 256), lambda i: (i, 0, 0)),
        ],
        out_shape=[
            jax.ShapeDtypeStruct((B_K, NUM_GROUP_K, N_K), jnp.float32),
            jax.ShapeDtypeStruct((B_K, NUM_GROUP_K, 256), jnp.float32),
        ],
        compiler_params=pltpu.CompilerParams(
            dimension_semantics=("parallel",),
        ),
        interpret=interpret,
    )(xt[:, :, None, :], center)


from jax import lax
from jax.experimental.pallas import tpu_sc as plsc

_CAP = 4096
_BIG = 1 << 30
_INF = float("inf")


def _iota16():
    return lax.broadcasted_iota(jnp.int32, (16,), 0)


def _splat(v, dtype=jnp.int32):
    return jnp.full((16,), v, dtype=dtype)


def _bcast_min(v, scr):
    for s in (8, 4, 2, 1):
        scr[...] = v
        rot = plsc.load_gather(scr, [(_iota16() + s) & 15])
        v = jnp.minimum(v, rot)
    return v


def _sc_group_body(
    d2_hbm, v256_hbm, xq_hbm, ctr_hbm, out_hbm,
    d2bufA, d2bufB, vbufA, vbufB, xqbuf, ctrbuf,
    scrf0, scri0, scrf1, scri1, selidsA, selidsB, outbuf,
    sem0, sem1, sem2, sem3,
):
    cid = lax.axis_index("c")
    sid = lax.axis_index("s")
    wid = sid * 2 + cid
    b = wid // 2
    half = wid % 2
    row0 = half * 128

    pltpu.sync_copy(xq_hbm.at[b], xqbuf)
    coff = pl.multiple_of(half * 384, 384)
    pltpu.sync_copy(ctr_hbm.at[b, pl.ds(coff, 384)], ctrbuf)

    sems = (sem0, sem1, sem2, sem3)
    dbufs = (d2bufA, d2bufB)
    vbufs = (vbufA, vbufB)

    def issue(r, side, bk):
        sem = sems[2 * bk + side]
        pltpu.async_copy(d2_hbm.at[b, row0 + r], dbufs[side].at[bk], sem)
        pltpu.async_copy(v256_hbm.at[b, row0 + r], vbufs[side].at[bk], sem)

    def wait(r, side, bk):
        sem = sems[2 * bk + side]
        pltpu.make_async_copy(
            d2_hbm.at[b, row0 + r], dbufs[side].at[bk], sem
        ).wait()
        pltpu.make_async_copy(
            v256_hbm.at[b, row0 + r], vbufs[side].at[bk], sem
        ).wait()

    def process_pair(r0, bk):
        lane0 = _iota16() == 0
        inf16 = _splat(_INF, jnp.float32)
        rows = ((r0, 0), (r0 + 1, 1))

        def one_row(k, r, db, vb, selids, p, scrf, scri):
            slot16 = _splat(bk)
            cm = [vb[bk, pl.ds(16 * j, 16)] for j in range(16)]
            vm = cm[0]
            for j in range(1, 16):
                vm = jnp.minimum(vm, cm[j])
            m = _bcast_min(vm, scrf)

            acc = _splat(_BIG)
            for j in range(16):
                acc = jnp.minimum(
                    acc, jnp.where(cm[j] == m, _iota16() + 16 * j, _BIG)
                )
            cls = _bcast_min(acc, scri)

            idxs = [cls + 256 * (_iota16() + 16 * t) for t in range(2)]
            mvs = [plsc.load_gather(db, [slot16, ix]) for ix in idxs]
            p_acc = jnp.where(mvs[0] == m, idxs[0], _BIG)
            p_acc = jnp.minimum(
                p_acc, jnp.where(mvs[1] == m, idxs[1], _BIG)
            )
            pid = _bcast_min(p_acc, scri)

            plsc.store_scatter(selids, [_splat(k)], pid, mask=lane0)
            plsc.store_scatter(db, [slot16, pid], inf16, mask=lane0)
            nmv = jnp.where(idxs[0] == pid, inf16, mvs[0])
            nmv = jnp.minimum(nmv, jnp.where(idxs[1] == pid, inf16, mvs[1]))
            nm = _bcast_min(nmv, scrf)
            plsc.store_scatter(vb, [slot16, cls], nm, mask=lane0)

        def sel_body(k, _):
            for (r, p), db, vb, sel, sf, si in zip(
                rows, dbufs, vbufs, (selidsA, selidsB),
                (scrf0, scrf1), (scri0, scri1),
            ):
                one_row(k, r, db, vb, sel, p, sf, si)
            return 0

        lax.fori_loop(0, GROUP_SIZE_K, sel_body, 0)

        for (r, p), sel in zip(rows, (selidsA, selidsB)):
            for kk in range(2):
                kid = sel[pl.ds(16 * kk, 16)]
                for ch in range(6):
                    val = plsc.load_gather(xqbuf, [_splat(ch), kid])
                    if ch < 3:
                        ctr = plsc.load_gather(ctrbuf, [_splat(r * 3 + ch)])
                        val = val - ctr
                    pos = (_iota16() + 16 * kk) * 6 + ch
                    plsc.store_scatter(outbuf, [_splat(p), pos], val)
        pltpu.sync_copy(outbuf, out_hbm.at[b, pl.ds(row0 + r0, 2)])

    def issue_pair(r0, bank):
        issue(r0, 0, bank)
        issue(r0 + 1, 1, bank)

    def wait_pair(r0, bank):
        wait(r0, 0, bank)
        wait(r0 + 1, 1, bank)

    issue_pair(0, 0)

    def iter2(i, _):
        r0 = 4 * i
        issue_pair(r0 + 2, 1)
        wait_pair(r0, 0)
        process_pair(r0, 0)

        @pl.when(r0 + 4 < 128)
        def _():
            issue_pair(r0 + 4, 0)

        wait_pair(r0 + 2, 1)
        process_pair(r0 + 2, 1)
        return 0

    lax.fori_loop(0, 32, iter2, 0)


_SC_SCRATCH = [
    pltpu.VMEM((2, N_K), jnp.float32),
    pltpu.VMEM((2, N_K), jnp.float32),
    pltpu.VMEM((2, 256), jnp.float32),
    pltpu.VMEM((2, 256), jnp.float32),
    pltpu.VMEM((6, N_K), jnp.float32),
    pltpu.VMEM((384,), jnp.float32),
    pltpu.VMEM((16,), jnp.float32),
    pltpu.VMEM((16,), jnp.int32),
    pltpu.VMEM((16,), jnp.float32),
    pltpu.VMEM((16,), jnp.int32),
    pltpu.VMEM((32,), jnp.int32),
    pltpu.VMEM((32,), jnp.int32),
    pltpu.VMEM((2, 192), jnp.float32),
    pltpu.SemaphoreType.DMA,
    pltpu.SemaphoreType.DMA,
    pltpu.SemaphoreType.DMA,
    pltpu.SemaphoreType.DMA,
]


@functools.lru_cache(maxsize=None)
def _sc_group():
    return pl.kernel(
        _sc_group_body,
        out_type=jax.ShapeDtypeStruct((B_K, NUM_GROUP_K, 192), jnp.float32),
        mesh=plsc.VectorSubcoreMesh(core_axis_name="c", subcore_axis_name="s"),
        scratch_types=_SC_SCRATCH,
        compiler_params=pltpu.CompilerParams(needs_layout_passes=False),
    )


def kernel(xyz):
    B, N, C = xyz.shape
    xyz_only = xyz[:, :, :3]

    xt = jnp.transpose(xyz_only, (2, 0, 1))
    ct = _fps_centers(xt)
    center = jnp.transpose(ct, (1, 2, 0))

    d2, v256 = _d2_v256(xt, center)

    xq = jnp.transpose(xyz, (0, 2, 1))
    ctr_flat = center.reshape(B, NUM_GROUP_K * 3)
    neigh = _sc_group()(d2, v256, xq, ctr_flat)
    neighborhood = neigh.reshape(B, NUM_GROUP_K, GROUP_SIZE_K, 6)
    return (neighborhood, center)
